# Initial kernel scaffold; baseline (speedup 1.0000x reference)
import jax
import jax.numpy as jnp
from jax.experimental import pallas as pl


def kernel(x, edge_index, edge_attr, W_in, b_in, W1, b1, W2, b2, W3, b3):
    raise NotImplementedError("write your pallas kernel here")


# speedup vs baseline: 7.0591x; 7.0591x over previous
import functools

import jax
import jax.numpy as jnp
from jax import lax
from jax.experimental import pallas as pl
from jax.experimental.pallas import tpu as pltpu
from jax.experimental.pallas import tpu_sc as plsc

N_NODES = 10000
N_EDGES = 320000
D_IN = 128
D_H = 64

NC = 2
NS = 16
NW = NC * NS
EPW = N_EDGES // NW
BLK = 80
NBLK = EPW // BLK
ROWS_PER_TILE = 624
ROWS_TAIL = N_NODES - NS * ROWS_PER_TILE

_mesh = plsc.VectorSubcoreMesh(core_axis_name="c", subcore_axis_name="s")


@functools.partial(
    pl.kernel,
    out_type=jax.ShapeDtypeStruct((NC, N_NODES, D_H), jnp.float32),
    mesh=_mesh,
    compiler_params=pltpu.CompilerParams(use_tc_tiling_on_sc=False),
    scratch_types=[
        pltpu.VMEM((NBLK, BLK), jnp.int32),
        pltpu.VMEM((NBLK, BLK), jnp.int32),
        pltpu.VMEM((NBLK, BLK), jnp.float32),
        pltpu.VMEM((BLK, D_H), jnp.float32),
        pltpu.VMEM_SHARED((N_NODES, D_H), jnp.float32),
    ],
)
def _sc_messages(h_hbm, src_hbm, tgt_hbm, attr_hbm, zeros_hbm, out_hbm,
                 src_v, tgt_v, attr_v, rows_v, acc_sh):
    cid = lax.axis_index("c")
    sid = lax.axis_index("s")
    wid = cid * NS + sid

    pltpu.sync_copy(src_hbm.at[wid], src_v)
    pltpu.sync_copy(tgt_hbm.at[wid], tgt_v)
    pltpu.sync_copy(attr_hbm.at[wid], attr_v)

    r0 = sid * ROWS_PER_TILE
    pltpu.sync_copy(zeros_hbm.at[pl.ds(r0, ROWS_PER_TILE)],
                    acc_sh.at[pl.ds(r0, ROWS_PER_TILE)])

    @pl.when(sid == 0)
    def _zero_tail():
        t0 = NS * ROWS_PER_TILE
        pltpu.sync_copy(zeros_hbm.at[pl.ds(t0, ROWS_TAIL)],
                        acc_sh.at[pl.ds(t0, ROWS_TAIL)])

    plsc.subcore_barrier()

    def body(j, carry):
        pltpu.sync_copy(h_hbm.at[src_v.at[j]], rows_v)

        for e0 in range(0, BLK, 16):
            a = attr_v[j, pl.ds(e0, 16)]
            for i in range(16):
                sv = jnp.full((16,), a[i], jnp.float32)
                e = e0 + i
                for k in range(D_H // 16):
                    rows_v[e, pl.ds(k * 16, 16)] = (
                        rows_v[e, pl.ds(k * 16, 16)] * sv)

        pltpu.sync_copy(rows_v, acc_sh.at[tgt_v.at[j]], add=True)
        return carry

    lax.fori_loop(0, NBLK, body, 0)

    plsc.subcore_barrier()
    pltpu.sync_copy(acc_sh.at[pl.ds(r0, ROWS_PER_TILE)],
                    out_hbm.at[cid, pl.ds(r0, ROWS_PER_TILE)])

    @pl.when(sid == 0)
    def _write_tail():
        t0 = NS * ROWS_PER_TILE
        pltpu.sync_copy(acc_sh.at[pl.ds(t0, ROWS_TAIL)],
                        out_hbm.at[cid, pl.ds(t0, ROWS_TAIL)])


def _tc_in_body(x_ref, w_ref, b_ref, o_ref):
    o_ref[...] = jnp.maximum(
        jnp.dot(x_ref[...], w_ref[...], preferred_element_type=jnp.float32)
        + b_ref[...], 0.0)


_tc_in = pl.pallas_call(
    _tc_in_body,
    out_shape=jax.ShapeDtypeStruct((N_NODES, D_H), jnp.float32),
)


def _tc_layer_body(h_ref, m_ref, w_ref, b_ref, o_ref):
    t = h_ref[...] + m_ref[0] + m_ref[1]
    o_ref[...] = jnp.maximum(
        jnp.dot(t, w_ref[...], preferred_element_type=jnp.float32)
        + b_ref[...], 0.0)


_tc_layer = pl.pallas_call(
    _tc_layer_body,
    out_shape=jax.ShapeDtypeStruct((N_NODES, D_H), jnp.float32),
)


def kernel(x, edge_index, edge_attr, W_in, b_in, W1, b1, W2, b2, W3, b3):
    src = edge_index[0].astype(jnp.int32).reshape(NW, NBLK, BLK)
    tgt = edge_index[1].astype(jnp.int32).reshape(NW, NBLK, BLK)
    attr = edge_attr.astype(jnp.float32).reshape(NW, NBLK, BLK)
    zeros = jnp.zeros((N_NODES, D_H), jnp.float32)

    h = _tc_in(x, W_in, b_in.reshape(1, D_H))
    states = [h]
    for (Wl, bl) in ((W1, b1), (W2, b2), (W3, b3)):
        m = _sc_messages(h, src, tgt, attr, zeros)
        h = _tc_layer(h, m, Wl, bl.reshape(1, D_H))
        states.append(h)
    return tuple(states)


# speedup vs baseline: 12.1161x; 1.7164x over previous
import functools

import jax
import jax.numpy as jnp
from jax import lax
from jax.experimental import pallas as pl
from jax.experimental.pallas import tpu as pltpu
from jax.experimental.pallas import tpu_sc as plsc

N_NODES = 10000
N_EDGES = 320000
D_IN = 128
D_H = 64
NV = D_H // 16

NC = 2
NS = 16
NW = NC * NS
EPW = N_EDGES // NW
BLK = 80
NBLK = EPW // BLK
ROWS_PER_TILE = 624
ROWS_TAIL = N_NODES - NS * ROWS_PER_TILE

_mesh = plsc.VectorSubcoreMesh(core_axis_name="c", subcore_axis_name="s")


@functools.partial(
    pl.kernel,
    out_type=jax.ShapeDtypeStruct((NC, N_NODES, D_H), jnp.float32),
    mesh=_mesh,
    compiler_params=pltpu.CompilerParams(use_tc_tiling_on_sc=False),
    scratch_types=[
        pltpu.VMEM((NBLK, BLK), jnp.int32),
        pltpu.VMEM((NBLK, BLK), jnp.int32),
        pltpu.VMEM((NBLK, BLK), jnp.float32),
        pltpu.VMEM((2, BLK, D_H), jnp.float32),
        pltpu.VMEM((2, BLK, D_H), jnp.float32),
        pltpu.VMEM_SHARED((N_NODES, D_H), jnp.float32),
        pltpu.SemaphoreType.DMA,
        pltpu.SemaphoreType.DMA,
        pltpu.SemaphoreType.DMA,
        pltpu.SemaphoreType.DMA,
    ],
)
def _sc_messages(h_hbm, src_hbm, tgt_hbm, attr_hbm, zeros_hbm, out_hbm,
                 src_v, tgt_v, attr_v, gbuf, sbuf, acc_sh,
                 sem_g0, sem_g1, sem_s0, sem_s1):
    cid = lax.axis_index("c")
    sid = lax.axis_index("s")
    wid = cid * NS + sid
    sem_g = (sem_g0, sem_g1)
    sem_s = (sem_s0, sem_s1)

    pltpu.sync_copy(src_hbm.at[wid], src_v)
    pltpu.sync_copy(tgt_hbm.at[wid], tgt_v)
    pltpu.sync_copy(attr_hbm.at[wid], attr_v)

    r0 = sid * ROWS_PER_TILE
    pltpu.sync_copy(zeros_hbm.at[pl.ds(r0, ROWS_PER_TILE)],
                    acc_sh.at[pl.ds(r0, ROWS_PER_TILE)])

    @pl.when(sid == 0)
    def _zero_tail():
        t0 = NS * ROWS_PER_TILE
        pltpu.sync_copy(zeros_hbm.at[pl.ds(t0, ROWS_TAIL)],
                        acc_sh.at[pl.ds(t0, ROWS_TAIL)])

    plsc.subcore_barrier()

    def start_gather(j, b):
        return pltpu.async_copy(h_hbm.at[src_v.at[j]], gbuf.at[b], sem_g[b])

    def wait_gather(j, b):
        pltpu.make_async_copy(h_hbm.at[src_v.at[j]], gbuf.at[b], sem_g[b]).wait()

    def start_scatter(j, b):
        return pltpu.async_copy(sbuf.at[b], acc_sh.at[tgt_v.at[j]], sem_s[b],
                                add=True)

    def wait_scatter(j, b):
        pltpu.make_async_copy(sbuf.at[b], acc_sh.at[tgt_v.at[j]],
                              sem_s[b]).wait()

    def scale(j, b):
        def grp(g, c):
            a16 = attr_v[j, pl.ds(g * 16, 16)]
            for i in range(16):
                sv = jnp.full((16,), a16[i], jnp.float32)
                e = g * 16 + i
                for k in range(NV):
                    sbuf[b, e, pl.ds(k * 16, 16)] = (
                        gbuf[b, e, pl.ds(k * 16, 16)] * sv)
            return c
        lax.fori_loop(0, BLK // 16, grp, 0)

    g0 = start_gather(0, 0)
    g1 = start_gather(1, 1)
    g0.wait()
    scale(0, 0)
    s0 = start_scatter(0, 0)
    g0 = start_gather(2, 0)
    g1.wait()
    scale(1, 1)
    s1 = start_scatter(1, 1)
    g1 = start_gather(3, 1)
    g0.wait()
    s0.wait()
    scale(2, 0)
    start_scatter(2, 0)
    start_gather(4, 0)

    def body(t, carry):
        a = 3 + 2 * t
        wait_gather(a, 1)
        wait_scatter(a - 2, 1)
        scale(a, 1)
        start_scatter(a, 1)

        @pl.when(a + 2 < NBLK)
        def _():
            start_gather(a + 2, 1)

        wait_gather(a + 1, 0)
        wait_scatter(a - 1, 0)
        scale(a + 1, 0)
        start_scatter(a + 1, 0)

        @pl.when(a + 3 < NBLK)
        def _():
            start_gather(a + 3, 0)
        return carry

    lax.fori_loop(0, (NBLK - 3) // 2, body, 0)

    wait_scatter(NBLK - 2, 1)
    wait_scatter(NBLK - 1, 0)

    plsc.subcore_barrier()
    pltpu.sync_copy(acc_sh.at[pl.ds(r0, ROWS_PER_TILE)],
                    out_hbm.at[cid, pl.ds(r0, ROWS_PER_TILE)])

    @pl.when(sid == 0)
    def _write_tail():
        t0 = NS * ROWS_PER_TILE
        pltpu.sync_copy(acc_sh.at[pl.ds(t0, ROWS_TAIL)],
                        out_hbm.at[cid, pl.ds(t0, ROWS_TAIL)])


def _tc_in_body(x_ref, w_ref, b_ref, o_ref):
    o_ref[...] = jnp.maximum(
        jnp.dot(x_ref[...], w_ref[...], preferred_element_type=jnp.float32)
        + b_ref[...], 0.0)


_tc_in = pl.pallas_call(
    _tc_in_body,
    out_shape=jax.ShapeDtypeStruct((N_NODES, D_H), jnp.float32),
)


def _tc_layer_body(h_ref, m_ref, w_ref, b_ref, o_ref):
    t = h_ref[...] + m_ref[0] + m_ref[1]
    o_ref[...] = jnp.maximum(
        jnp.dot(t, w_ref[...], preferred_element_type=jnp.float32)
        + b_ref[...], 0.0)


_tc_layer = pl.pallas_call(
    _tc_layer_body,
    out_shape=jax.ShapeDtypeStruct((N_NODES, D_H), jnp.float32),
)


def kernel(x, edge_index, edge_attr, W_in, b_in, W1, b1, W2, b2, W3, b3):
    src = edge_index[0].astype(jnp.int32).reshape(NW, NBLK, BLK)
    tgt = edge_index[1].astype(jnp.int32).reshape(NW, NBLK, BLK)
    attr = edge_attr.astype(jnp.float32).reshape(NW, NBLK, BLK)
    zeros = jnp.zeros((N_NODES, D_H), jnp.float32)

    h = _tc_in(x, W_in, b_in.reshape(1, D_H))
    states = [h]
    for (Wl, bl) in ((W1, b1), (W2, b2), (W3, b3)):
        m = _sc_messages(h, src, tgt, attr, zeros)
        h = _tc_layer(h, m, Wl, bl.reshape(1, D_H))
        states.append(h)
    return tuple(states)
